# baseline (speedup 1.0000x reference)
import functools

import jax
import jax.numpy as jnp
from jax.experimental import pallas as pl
from jax.experimental.pallas import tpu as pltpu

_B = 8
_PAD = 128


def _conv_pool_kernel(h, w, x_ref, mwrap_ref, pmask_ref, wk_ref, bc_ref,
                      pooled_ref, scr_ref):
    hw = h * w
    mask_l = mwrap_ref[0:1, :]
    mask_r = mwrap_ref[1:2, :]

    convs = []
    for b in range(_B):
        xb = x_ref[b].astype(jnp.bfloat16)
        scr_ref[b, :, :_PAD] = jnp.zeros_like(scr_ref[b, :, :_PAD])
        scr_ref[b, :, _PAD + hw:] = jnp.zeros_like(scr_ref[b, :, _PAD + hw:])
        scr_ref[b, :, _PAD:_PAD + hw] = xb

        taps = []
        for di in range(3):
            for dj in range(3):
                off = (di - 1) * w + (dj - 1)
                v = scr_ref[b, :, _PAD + off:_PAD + off + hw]
                if dj == 0:
                    v = v * mask_l
                elif dj == 2:
                    v = v * mask_r
                taps.append(v)
        patches_t = jnp.concatenate(taps, axis=0)

        conv = jax.lax.dot_general(
            patches_t, wk_ref[...], (((0,), (0,)), ((), ())),
            preferred_element_type=jnp.float32)
        convs.append(jnp.maximum(conv + bc_ref[...], 0.0))

    convcat = jnp.concatenate(convs, axis=0)
    pooled_ref[...] = jnp.dot(pmask_ref[...], convcat,
                              preferred_element_type=jnp.float32)


def _head_kernel(p_ref, w1_ref, b1_ref, w2_ref, b2_ref, logits_ref, feat_ref):
    emb = jnp.maximum(
        jnp.dot(p_ref[...].astype(jnp.bfloat16), w1_ref[...],
                preferred_element_type=jnp.float32) + b1_ref[...], 0.0)
    feat_ref[...] = emb
    logits_ref[...] = (jnp.dot(emb.astype(jnp.bfloat16), w2_ref[...],
                               preferred_element_type=jnp.float32)
                       + b2_ref[...])


@jax.jit
def _forward(x, wk, bc, w1, b1, w2, b2):
    n, c, h, w = x.shape
    hw = h * w

    fpad = wk.shape[-1]
    epad = w1.shape[-1]
    cpad = w2.shape[-1]

    x3 = x.reshape(n, c, hw)

    e = jnp.arange(hw)
    mwrap = jnp.concatenate([
        jnp.where(e % w != 0, 1.0, 0.0).reshape(1, hw),
        jnp.where(e % w != w - 1, 1.0, 0.0).reshape(1, hw),
        jnp.ones((6, hw), jnp.float32),
    ], axis=0).astype(jnp.bfloat16)

    pmask = jnp.kron(jnp.eye(_B, dtype=jnp.float32),
                     jnp.full((1, hw), 1.0 / hw, jnp.float32))

    pooled = pl.pallas_call(
        functools.partial(_conv_pool_kernel, h, w),
        out_shape=jax.ShapeDtypeStruct((n, fpad), jnp.float32),
        grid=(n // _B,),
        in_specs=[
            pl.BlockSpec((_B, c, hw), lambda i: (i, 0, 0)),
            pl.BlockSpec((8, hw), lambda i: (0, 0)),
            pl.BlockSpec((_B, _B * hw), lambda i: (0, 0)),
            pl.BlockSpec((9 * c, fpad), lambda i: (0, 0)),
            pl.BlockSpec((1, fpad), lambda i: (0, 0)),
        ],
        out_specs=pl.BlockSpec((_B, fpad), lambda i: (i, 0)),
        scratch_shapes=[pltpu.VMEM((_B, c, 2 * _PAD + hw), jnp.bfloat16)],
        compiler_params=pltpu.CompilerParams(
            dimension_semantics=("parallel",),
            vmem_limit_bytes=64 * 1024 * 1024,
        ),
    )(x3, mwrap, pmask, wk, bc)

    bm = n // 2
    logits_pad, feat_pad = pl.pallas_call(
        _head_kernel,
        out_shape=(
            jax.ShapeDtypeStruct((n, cpad), jnp.float32),
            jax.ShapeDtypeStruct((n, epad), jnp.float32),
        ),
        grid=(2,),
        in_specs=[
            pl.BlockSpec((bm, fpad), lambda i: (i, 0)),
            pl.BlockSpec((fpad, epad), lambda i: (0, 0)),
            pl.BlockSpec((1, epad), lambda i: (0, 0)),
            pl.BlockSpec((epad, cpad), lambda i: (0, 0)),
            pl.BlockSpec((1, cpad), lambda i: (0, 0)),
        ],
        out_specs=(
            pl.BlockSpec((bm, cpad), lambda i: (i, 0)),
            pl.BlockSpec((bm, epad), lambda i: (i, 0)),
        ),
        compiler_params=pltpu.CompilerParams(
            dimension_semantics=("parallel",),
        ),
    )(pooled, w1, b1, w2, b2)

    return logits_pad[:, :1000], feat_pad[:, :256]


def kernel(x, wk, bc, w1, b1, w2, b2):
    return _forward(x, wk, bc, w1, b1, w2, b2)


# speedup vs baseline: 1.1760x; 1.1760x over previous
import functools

import jax
import jax.numpy as jnp
from jax.experimental import pallas as pl
from jax.experimental.pallas import tpu as pltpu

_B = 8


def _conv_pool_kernel(h, w, x_ref, pmask_ref, wk_ref, bc_ref, pooled_ref):
    wp2 = w + 2
    rows = h * wp2
    span = rows + 2 * wp2

    convs = []
    for b in range(_B):
        xb = x_ref[b]
        win = jnp.concatenate([xb[:, :64], xb[:, 64:]], axis=0)
        wincat = jnp.concatenate([win[dj:dj + span, :] for dj in range(3)],
                                 axis=-1)
        patches = jnp.concatenate(
            [wincat[di * wp2:di * wp2 + rows, :] for di in range(3)],
            axis=-1)
        conv = jnp.dot(patches, wk_ref[...],
                       preferred_element_type=jnp.float32)
        convs.append(jnp.maximum(conv + bc_ref[...], 0.0))

    convcat = jnp.concatenate(convs, axis=0)
    pooled_ref[...] = jnp.dot(pmask_ref[...], convcat,
                              preferred_element_type=jnp.float32)


def _head_kernel(p_ref, w1_ref, b1_ref, w2_ref, b2_ref, logits_ref, feat_ref):
    emb = jnp.maximum(
        jnp.dot(p_ref[...].astype(jnp.bfloat16), w1_ref[...],
                preferred_element_type=jnp.float32) + b1_ref[...], 0.0)
    feat_ref[...] = emb
    logits_ref[...] = (jnp.dot(emb.astype(jnp.bfloat16), w2_ref[...],
                               preferred_element_type=jnp.float32)
                       + b2_ref[...])


@jax.jit
def _forward(x, wk, bc, w1, b1, w2, b2):
    n, c, h, w = x.shape
    wp2 = w + 2
    rows = h * wp2
    rows_pad = (h + 4) * wp2

    fpad = wk.shape[-1]
    epad = w1.shape[-1]
    cpad = w2.shape[-1]

    x_nhwc = jnp.transpose(x, (0, 2, 3, 1))
    x_pad = jnp.pad(x_nhwc, ((0, 0), (1, 3), (1, 1), (0, 0))).astype(jnp.bfloat16)
    x960 = x_pad.reshape(n, rows_pad, c)
    x_flat = jnp.concatenate(
        [x960[:, :rows_pad // 2, :], x960[:, rows_pad // 2:, :]], axis=-1)

    base = jnp.where(jnp.arange(rows) % wp2 < w, 1.0 / (h * w), 0.0)
    pmask = jnp.kron(jnp.eye(_B, dtype=jnp.float32),
                     base.astype(jnp.float32).reshape(1, rows))

    pooled = pl.pallas_call(
        functools.partial(_conv_pool_kernel, h, w),
        out_shape=jax.ShapeDtypeStruct((n, fpad), jnp.float32),
        grid=(n // _B,),
        in_specs=[
            pl.BlockSpec((_B, rows_pad // 2, 2 * c), lambda i: (i, 0, 0)),
            pl.BlockSpec((_B, _B * rows), lambda i: (0, 0)),
            pl.BlockSpec((9 * c, fpad), lambda i: (0, 0)),
            pl.BlockSpec((1, fpad), lambda i: (0, 0)),
        ],
        out_specs=pl.BlockSpec((_B, fpad), lambda i: (i, 0)),
        compiler_params=pltpu.CompilerParams(
            dimension_semantics=("parallel",),
            vmem_limit_bytes=64 * 1024 * 1024,
        ),
    )(x_flat, pmask, wk, bc)

    bm = n // 2
    logits_pad, feat_pad = pl.pallas_call(
        _head_kernel,
        out_shape=(
            jax.ShapeDtypeStruct((n, cpad), jnp.float32),
            jax.ShapeDtypeStruct((n, epad), jnp.float32),
        ),
        grid=(2,),
        in_specs=[
            pl.BlockSpec((bm, fpad), lambda i: (i, 0)),
            pl.BlockSpec((fpad, epad), lambda i: (0, 0)),
            pl.BlockSpec((1, epad), lambda i: (0, 0)),
            pl.BlockSpec((epad, cpad), lambda i: (0, 0)),
            pl.BlockSpec((1, cpad), lambda i: (0, 0)),
        ],
        out_specs=(
            pl.BlockSpec((bm, cpad), lambda i: (i, 0)),
            pl.BlockSpec((bm, epad), lambda i: (i, 0)),
        ),
        compiler_params=pltpu.CompilerParams(
            dimension_semantics=("parallel",),
        ),
    )(pooled, w1, b1, w2, b2)

    return logits_pad[:, :1000], feat_pad[:, :256]


def kernel(x, wk, bc, w1, b1, w2, b2):
    return _forward(x, wk, bc, w1, b1, w2, b2)


# speedup vs baseline: 1.4756x; 1.2547x over previous
import functools

import jax
import jax.numpy as jnp
from jax.experimental import pallas as pl
from jax.experimental.pallas import tpu as pltpu

_B = 8


def _conv_pool_kernel(h, w, n_steps, x_hbm, pmask_ref, wk_ref, bc_ref,
                      pooled_ref, x_buf, in_sem):
    wp2 = w + 2
    rows = h * wp2
    span = rows + 2 * wp2

    def dma_in(slot, step):
        pltpu.make_async_copy(x_hbm.at[pl.ds(step * _B, _B)],
                              x_buf.at[slot], in_sem.at[slot]).start()

    def wait_in(slot):
        pltpu.make_async_copy(x_hbm.at[pl.ds(0, _B)],
                              x_buf.at[slot], in_sem.at[slot]).wait()

    dma_in(0, 0)

    def body(step, _):
        cur = jax.lax.rem(step, 2)
        nxt = jax.lax.rem(step + 1, 2)

        @pl.when(step + 1 < n_steps)
        def _():
            dma_in(nxt, step + 1)

        wait_in(cur)

        convs = []
        for b in range(_B):
            win = x_buf[cur, b]
            wincat = jnp.concatenate(
                [win[dj:dj + span, :] for dj in range(3)], axis=-1)
            patches = jnp.concatenate(
                [wincat[di * wp2:di * wp2 + rows, :] for di in range(3)],
                axis=-1)
            conv = jnp.dot(patches, wk_ref[...],
                           preferred_element_type=jnp.float32)
            convs.append(jnp.maximum(conv + bc_ref[...], 0.0))

        convcat = jnp.concatenate(convs, axis=0)
        pooled_ref[pl.ds(step * _B, _B), :] = jnp.dot(
            pmask_ref[...], convcat,
            preferred_element_type=jnp.float32)
        return ()

    jax.lax.fori_loop(0, n_steps, body, ())


def _head_kernel(p_ref, w1_ref, b1_ref, w2_ref, b2_ref, logits_ref, feat_ref):
    emb = jnp.maximum(
        jnp.dot(p_ref[...].astype(jnp.bfloat16), w1_ref[...],
                preferred_element_type=jnp.float32) + b1_ref[...], 0.0)
    feat_ref[...] = emb
    logits_ref[...] = (jnp.dot(emb.astype(jnp.bfloat16), w2_ref[...],
                               preferred_element_type=jnp.float32)
                       + b2_ref[...])


@jax.jit
def _forward(x, wk, bc, w1, b1, w2, b2):
    n, c, h, w = x.shape
    wp2 = w + 2
    rows = h * wp2
    rows_pad = (h + 4) * wp2

    fpad = wk.shape[-1]
    epad = w1.shape[-1]
    cpad = w2.shape[-1]

    x_nhwc = jnp.transpose(x, (0, 2, 3, 1))
    x_pad = jnp.pad(x_nhwc, ((0, 0), (1, 3), (1, 1), (0, 0))).astype(jnp.bfloat16)
    x_flat = x_pad.reshape(n, rows_pad, c)

    base = jnp.where(jnp.arange(rows) % wp2 < w, 1.0 / (h * w), 0.0)
    pmask = jnp.kron(jnp.eye(_B, dtype=jnp.float32),
                     base.astype(jnp.float32).reshape(1, rows))

    n_steps = n // _B
    pooled = pl.pallas_call(
        functools.partial(_conv_pool_kernel, h, w, n_steps),
        out_shape=jax.ShapeDtypeStruct((n, fpad), jnp.float32),
        in_specs=[
            pl.BlockSpec(memory_space=pl.ANY),
            pl.BlockSpec(memory_space=pltpu.MemorySpace.VMEM),
            pl.BlockSpec(memory_space=pltpu.MemorySpace.VMEM),
            pl.BlockSpec(memory_space=pltpu.MemorySpace.VMEM),
        ],
        out_specs=pl.BlockSpec(memory_space=pltpu.MemorySpace.VMEM),
        scratch_shapes=[
            pltpu.VMEM((2, _B, rows_pad, c), jnp.bfloat16),
            pltpu.SemaphoreType.DMA((2,)),
        ],
        compiler_params=pltpu.CompilerParams(
            vmem_limit_bytes=64 * 1024 * 1024,
        ),
    )(x_flat, pmask, wk, bc)

    bm = n // 2
    logits_pad, feat_pad = pl.pallas_call(
        _head_kernel,
        out_shape=(
            jax.ShapeDtypeStruct((n, cpad), jnp.float32),
            jax.ShapeDtypeStruct((n, epad), jnp.float32),
        ),
        grid=(2,),
        in_specs=[
            pl.BlockSpec((bm, fpad), lambda i: (i, 0)),
            pl.BlockSpec((fpad, epad), lambda i: (0, 0)),
            pl.BlockSpec((1, epad), lambda i: (0, 0)),
            pl.BlockSpec((epad, cpad), lambda i: (0, 0)),
            pl.BlockSpec((1, cpad), lambda i: (0, 0)),
        ],
        out_specs=(
            pl.BlockSpec((bm, cpad), lambda i: (i, 0)),
            pl.BlockSpec((bm, epad), lambda i: (i, 0)),
        ),
        compiler_params=pltpu.CompilerParams(
            dimension_semantics=("parallel",),
        ),
    )(pooled, w1, b1, w2, b2)

    return logits_pad[:, :1000], feat_pad[:, :256]


def kernel(x, wk, bc, w1, b1, w2, b2):
    return _forward(x, wk, bc, w1, b1, w2, b2)


# speedup vs baseline: 1.4801x; 1.0030x over previous
import functools

import jax
import jax.numpy as jnp
from jax.experimental import pallas as pl
from jax.experimental.pallas import tpu as pltpu

_B = 8


def _conv_pool_kernel(h, w, x_ref, pmask_ref, wk_ref, bc_ref, pooled_ref):
    wp2 = w + 2
    rows = h * wp2
    span = rows + 2 * wp2

    convs = []
    for b in range(_B):
        win = x_ref[b]
        wincat = jnp.concatenate([win[dj:dj + span, :] for dj in range(3)],
                                 axis=-1)
        patches = jnp.concatenate(
            [wincat[di * wp2:di * wp2 + rows, :] for di in range(3)],
            axis=-1)
        conv = jnp.dot(patches, wk_ref[...],
                       preferred_element_type=jnp.float32)
        convs.append(jnp.maximum(conv + bc_ref[...], 0.0))

    convcat = jnp.concatenate(convs, axis=0)
    pooled_ref[...] = jnp.dot(pmask_ref[...], convcat,
                              preferred_element_type=jnp.float32)


def _head_kernel(p_ref, w1_ref, b1_ref, w2_ref, b2_ref, logits_ref, feat_ref):
    emb = jnp.maximum(
        jnp.dot(p_ref[...].astype(jnp.bfloat16), w1_ref[...],
                preferred_element_type=jnp.float32) + b1_ref[...], 0.0)
    feat_ref[...] = emb
    logits_ref[...] = (jnp.dot(emb.astype(jnp.bfloat16), w2_ref[...],
                               preferred_element_type=jnp.float32)
                       + b2_ref[...])


@jax.jit
def _forward(x, wk, bc, w1, b1, w2, b2):
    n, c, h, w = x.shape
    wp2 = w + 2
    rows = h * wp2
    rows_pad = (h + 4) * wp2

    fpad = wk.shape[-1]
    epad = w1.shape[-1]
    cpad = w2.shape[-1]

    x_nhwc = jnp.transpose(x, (0, 2, 3, 1))
    x_pad = jnp.pad(x_nhwc, ((0, 0), (1, 3), (1, 1), (0, 0))).astype(jnp.bfloat16)
    x_flat = x_pad.reshape(n, rows_pad, c)

    base = jnp.where(jnp.arange(rows) % wp2 < w, 1.0 / (h * w), 0.0)
    pmask = jnp.kron(jnp.eye(_B, dtype=jnp.float32),
                     base.astype(jnp.float32).reshape(1, rows))

    pooled = pl.pallas_call(
        functools.partial(_conv_pool_kernel, h, w),
        out_shape=jax.ShapeDtypeStruct((n, fpad), jnp.float32),
        grid=(n // _B,),
        in_specs=[
            pl.BlockSpec((_B, rows_pad, c), lambda i: (i, 0, 0)),
            pl.BlockSpec((_B, _B * rows), lambda i: (0, 0)),
            pl.BlockSpec((9 * c, fpad), lambda i: (0, 0)),
            pl.BlockSpec((1, fpad), lambda i: (0, 0)),
        ],
        out_specs=pl.BlockSpec((_B, fpad), lambda i: (i, 0)),
        compiler_params=pltpu.CompilerParams(
            dimension_semantics=("parallel",),
            vmem_limit_bytes=64 * 1024 * 1024,
        ),
    )(x_flat, pmask, wk, bc)

    bm = n // 2
    logits_pad, feat_pad = pl.pallas_call(
        _head_kernel,
        out_shape=(
            jax.ShapeDtypeStruct((n, cpad), jnp.float32),
            jax.ShapeDtypeStruct((n, epad), jnp.float32),
        ),
        grid=(2,),
        in_specs=[
            pl.BlockSpec((bm, fpad), lambda i: (i, 0)),
            pl.BlockSpec((fpad, epad), lambda i: (0, 0)),
            pl.BlockSpec((1, epad), lambda i: (0, 0)),
            pl.BlockSpec((epad, cpad), lambda i: (0, 0)),
            pl.BlockSpec((1, cpad), lambda i: (0, 0)),
        ],
        out_specs=(
            pl.BlockSpec((bm, cpad), lambda i: (i, 0)),
            pl.BlockSpec((bm, epad), lambda i: (i, 0)),
        ),
        compiler_params=pltpu.CompilerParams(
            dimension_semantics=("parallel",),
        ),
    )(pooled, w1, b1, w2, b2)

    return logits_pad[:, :1000], feat_pad[:, :256]


def kernel(x, wk, bc, w1, b1, w2, b2):
    return _forward(x, wk, bc, w1, b1, w2, b2)
